# Initial kernel scaffold; baseline (speedup 1.0000x reference)
import jax
import jax.numpy as jnp
from jax.experimental import pallas as pl


def kernel(x, W1, b1, W2, b2, W3, b3, ln_w, ln_b, dn_w, dn_b, groups):
    raise NotImplementedError("write your pallas kernel here")


# speedup vs baseline: 4.9128x; 4.9128x over previous
import functools

import jax
import jax.numpy as jnp
from jax.experimental import pallas as pl
from jax.experimental.pallas import tpu as pltpu

D_MODEL = 512
N_SUB = 16
EPS = 1e-5
BLK = 512


def _gelu_exact(v):
    return 0.5 * v * (1.0 + jax.lax.erf(v * 0.7071067811865476))


def _fused_kernel(x_ref, w1_ref, b1_ref, w2_ref, b2_ref, w3_ref, b3_ref,
                  lnw_ref, lnb_ref, dnw_ref, dnb_ref, g_ref,
                  out_ref, logits_ref):
    x = x_ref[...]

    mean = jnp.mean(x, axis=-1, keepdims=True)
    xc = x - mean
    var = jnp.mean(xc * xc, axis=-1, keepdims=True)
    xhat = xc * jax.lax.rsqrt(var + EPS)

    g = g_ref[0]
    sub = jax.lax.broadcasted_iota(jnp.int32, (N_SUB, BLK), 0)
    oh = (g == sub).astype(jnp.float32)
    dnums = (((0,), (0,)), ((), ()))
    gamma = jax.lax.dot_general(oh, lnw_ref[...], dnums,
                                preferred_element_type=jnp.float32)
    beta = jax.lax.dot_general(oh, lnb_ref[...], dnums,
                               preferred_element_type=jnp.float32)
    valid = ((g >= 0) & (g < N_SUB))
    validf = valid.astype(jnp.float32)
    validc = jnp.transpose(validf)
    gamma = gamma * validc + dnw_ref[...] * (1.0 - validc)
    beta = beta * validc + dnb_ref[...] * (1.0 - validc)
    out_ref[...] = xhat * gamma + beta

    cdims = (((1,), (1,)), ((), ()))
    h = jax.lax.dot_general(x, w1_ref[...], cdims,
                            preferred_element_type=jnp.float32) + b1_ref[...]
    h = _gelu_exact(h)
    h = jax.lax.dot_general(h, w2_ref[...], cdims,
                            preferred_element_type=jnp.float32) + b2_ref[...]
    h = _gelu_exact(h)
    logits_ref[...] = jax.lax.dot_general(
        h, w3_ref[...], cdims, preferred_element_type=jnp.float32) + b3_ref[...]


@functools.partial(jax.jit, static_argnames=())
def kernel(x, W1, b1, W2, b2, W3, b3, ln_w, ln_b, dn_w, dn_b, groups):
    B = x.shape[0]
    nb = B // BLK
    g3 = groups.astype(jnp.int32).reshape(nb, 1, BLK)

    rep = lambda *shape: pl.BlockSpec(shape, lambda i: (0,) * len(shape))
    out, logits = pl.pallas_call(
        _fused_kernel,
        grid=(nb,),
        in_specs=[
            pl.BlockSpec((BLK, D_MODEL), lambda i: (i, 0)),
            rep(256, D_MODEL),
            rep(1, 256),
            rep(128, 256),
            rep(1, 128),
            rep(N_SUB, 128),
            rep(1, N_SUB),
            rep(N_SUB, D_MODEL),
            rep(N_SUB, D_MODEL),
            rep(1, D_MODEL),
            rep(1, D_MODEL),
            pl.BlockSpec((1, 1, BLK), lambda i: (i, 0, 0)),
        ],
        out_specs=[
            pl.BlockSpec((BLK, D_MODEL), lambda i: (i, 0)),
            pl.BlockSpec((BLK, N_SUB), lambda i: (i, 0)),
        ],
        out_shape=[
            jax.ShapeDtypeStruct((B, D_MODEL), jnp.float32),
            jax.ShapeDtypeStruct((B, N_SUB), jnp.float32),
        ],
        compiler_params=pltpu.CompilerParams(
            dimension_semantics=("parallel",)),
    )(x, W1, b1.reshape(1, 256), W2, b2.reshape(1, 128), W3,
      b3.reshape(1, N_SUB), ln_w, ln_b, dn_w.reshape(1, D_MODEL),
      dn_b.reshape(1, D_MODEL), g3)
    return (out, logits)


# speedup vs baseline: 6.2522x; 1.2726x over previous
import functools

import jax
import jax.numpy as jnp
from jax.experimental import pallas as pl
from jax.experimental.pallas import tpu as pltpu

D_MODEL = 512
N_SUB = 16
EPS = 1e-5
BLK = 1024


def _gelu_exact(v):
    return 0.5 * v * (1.0 + jax.lax.erf(v * 0.7071067811865476))


def _fused_kernel(x_ref, w1_ref, b1_ref, w2_ref, b2_ref, w3_ref, b3_ref,
                  tab_ref, g_ref, out_ref, logits_ref):
    x = x_ref[...]

    mean = jnp.mean(x, axis=-1, keepdims=True)
    xc = x - mean
    var = jnp.mean(xc * xc, axis=-1, keepdims=True)
    xhat = xc * jax.lax.rsqrt(var + EPS)

    g = g_ref[0]
    sub = jax.lax.broadcasted_iota(jnp.int32, (N_SUB + 1, BLK), 0)
    oh = (g == sub).astype(jnp.float32)
    gb = jax.lax.dot_general(oh, tab_ref[...], (((0,), (0,)), ((), ())),
                             preferred_element_type=jnp.float32)
    out_ref[...] = xhat * gb[:, :D_MODEL] + gb[:, D_MODEL:]

    cdims = (((1,), (1,)), ((), ()))
    h = jax.lax.dot_general(x, w1_ref[...], cdims,
                            preferred_element_type=jnp.float32) + b1_ref[...]
    h = _gelu_exact(h)
    h = jax.lax.dot_general(h, w2_ref[...], cdims,
                            preferred_element_type=jnp.float32) + b2_ref[...]
    h = _gelu_exact(h)
    logits_ref[...] = jax.lax.dot_general(
        h, w3_ref[...], cdims, preferred_element_type=jnp.float32) + b3_ref[...]


@functools.partial(jax.jit, static_argnames=())
def kernel(x, W1, b1, W2, b2, W3, b3, ln_w, ln_b, dn_w, dn_b, groups):
    B = x.shape[0]
    nb = B // BLK
    gi = groups.astype(jnp.int32)
    gi = jnp.where((gi >= 0) & (gi < N_SUB), gi, N_SUB).reshape(nb, 1, BLK)
    tab = jnp.concatenate(
        [jnp.concatenate([ln_w, dn_w[None, :]], axis=0),
         jnp.concatenate([ln_b, dn_b[None, :]], axis=0)], axis=1)

    rep = lambda *shape: pl.BlockSpec(shape, lambda i: (0,) * len(shape))
    out, logits = pl.pallas_call(
        _fused_kernel,
        grid=(nb,),
        in_specs=[
            pl.BlockSpec((BLK, D_MODEL), lambda i: (i, 0)),
            rep(256, D_MODEL),
            rep(1, 256),
            rep(128, 256),
            rep(1, 128),
            rep(N_SUB, 128),
            rep(1, N_SUB),
            rep(N_SUB + 1, 2 * D_MODEL),
            pl.BlockSpec((1, 1, BLK), lambda i: (i, 0, 0)),
        ],
        out_specs=[
            pl.BlockSpec((BLK, D_MODEL), lambda i: (i, 0)),
            pl.BlockSpec((BLK, N_SUB), lambda i: (i, 0)),
        ],
        out_shape=[
            jax.ShapeDtypeStruct((B, D_MODEL), jnp.float32),
            jax.ShapeDtypeStruct((B, N_SUB), jnp.float32),
        ],
        compiler_params=pltpu.CompilerParams(
            dimension_semantics=("parallel",)),
    )(x, W1, b1.reshape(1, 256), W2, b2.reshape(1, 128), W3,
      b3.reshape(1, N_SUB), tab, gi)
    return (out, logits)


# speedup vs baseline: 6.6211x; 1.0590x over previous
import functools

import jax
import jax.numpy as jnp
from jax.experimental import pallas as pl
from jax.experimental.pallas import tpu as pltpu

D_MODEL = 512
N_SUB = 16
EPS = 1e-5
BLK = 2048


def _gelu_exact(v):
    return 0.5 * v * (1.0 + jax.lax.erf(v * 0.7071067811865476))


def _fused_kernel(x_ref, w1_ref, b1_ref, w2_ref, b2_ref, w3_ref, b3_ref,
                  tab_ref, g_ref, out_ref, logits_ref):
    x = x_ref[...]

    mean = jnp.mean(x, axis=-1, keepdims=True)
    xc = x - mean
    var = jnp.mean(xc * xc, axis=-1, keepdims=True)
    xhat = xc * jax.lax.rsqrt(var + EPS)

    g = g_ref[0]
    sub = jax.lax.broadcasted_iota(jnp.int32, (N_SUB + 1, BLK), 0)
    oh = (g == sub).astype(jnp.float32)
    gb = jax.lax.dot_general(oh, tab_ref[...], (((0,), (0,)), ((), ())),
                             preferred_element_type=jnp.float32)
    out_ref[...] = xhat * gb[:, :D_MODEL] + gb[:, D_MODEL:]

    cdims = (((1,), (1,)), ((), ()))
    h = jax.lax.dot_general(x, w1_ref[...], cdims,
                            preferred_element_type=jnp.float32) + b1_ref[...]
    h = _gelu_exact(h)
    h = jax.lax.dot_general(h, w2_ref[...], cdims,
                            preferred_element_type=jnp.float32) + b2_ref[...]
    h = _gelu_exact(h)
    logits_ref[...] = jax.lax.dot_general(
        h, w3_ref[...], cdims, preferred_element_type=jnp.float32) + b3_ref[...]


@functools.partial(jax.jit, static_argnames=())
def kernel(x, W1, b1, W2, b2, W3, b3, ln_w, ln_b, dn_w, dn_b, groups):
    B = x.shape[0]
    nb = B // BLK
    gi = groups.astype(jnp.int32)
    gi = jnp.where((gi >= 0) & (gi < N_SUB), gi, N_SUB).reshape(nb, 1, BLK)
    tab = jnp.concatenate(
        [jnp.concatenate([ln_w, dn_w[None, :]], axis=0),
         jnp.concatenate([ln_b, dn_b[None, :]], axis=0)], axis=1)

    rep = lambda *shape: pl.BlockSpec(shape, lambda i: (0,) * len(shape))
    out, logits = pl.pallas_call(
        _fused_kernel,
        grid=(nb,),
        in_specs=[
            pl.BlockSpec((BLK, D_MODEL), lambda i: (i, 0)),
            rep(256, D_MODEL),
            rep(1, 256),
            rep(128, 256),
            rep(1, 128),
            rep(N_SUB, 128),
            rep(1, N_SUB),
            rep(N_SUB + 1, 2 * D_MODEL),
            pl.BlockSpec((1, 1, BLK), lambda i: (i, 0, 0)),
        ],
        out_specs=[
            pl.BlockSpec((BLK, D_MODEL), lambda i: (i, 0)),
            pl.BlockSpec((BLK, N_SUB), lambda i: (i, 0)),
        ],
        out_shape=[
            jax.ShapeDtypeStruct((B, D_MODEL), jnp.float32),
            jax.ShapeDtypeStruct((B, N_SUB), jnp.float32),
        ],
        compiler_params=pltpu.CompilerParams(
            dimension_semantics=("parallel",)),
    )(x, W1, b1.reshape(1, 256), W2, b2.reshape(1, 128), W3,
      b3.reshape(1, N_SUB), tab, gi)
    return (out, logits)
